# baseline (speedup 1.0000x reference)
import functools

import jax
import jax.numpy as jnp
from jax.experimental import pallas as pl
from jax.experimental.pallas import tpu as pltpu
from jax.experimental.pallas import tpu_sc as plsc

_K_TOP = 128
_LOCAL_K = 64
_G = 8
_NEG = -3.0e38


def _score_body(qp_ref, w_ref, k_ref, kg_ref, idx_ref, rem_ref,
                *, S, D, local_k, n_top, G):
    n_nl = n_top - local_k
    f32 = jnp.float32
    bf = jnp.bfloat16
    big = jnp.int32(2147483647)

    lane = jax.lax.broadcasted_iota(jnp.int32, (G, S), 1)
    is_local = lane >= (S - local_k)
    row = jax.lax.broadcasted_iota(jnp.int32, (G, S), 0)
    lane64 = jax.lax.broadcasted_iota(jnp.int32, (G, n_nl), 1)
    oh_lane = jax.lax.broadcasted_iota(jnp.int32, (n_nl, S), 1)

    kb = []
    scores = None
    for g in range(G):
        Wb = w_ref[g].astype(bf)
        kbg = k_ref[g].astype(bf)
        kp = jax.lax.dot_general(kbg, Wb, (((1,), (0,)), ((), ())),
                                 preferred_element_type=f32)
        qpg = qp_ref[g].astype(bf)
        sf = jax.lax.dot_general(qpg, kp.astype(bf), (((1,), (1,)), ((), ())),
                                 preferred_element_type=f32)
        scores = sf if g == 0 else jnp.where(row == g, sf, scores)
        kb.append(kbg)
        kg_ref[g, 0:local_k, :] = k_ref[g, S - local_k:S, :]

    scores = scores * (D ** -0.5)
    m8 = jnp.max(scores, axis=1, keepdims=True)
    e8 = jnp.exp(scores - m8)
    se8 = jnp.sum(e8, axis=1, keepdims=True)
    p_loc = jnp.sum(jnp.where(is_local, e8, 0.0), axis=1,
                    keepdims=True) / se8
    ms0 = jnp.where(is_local, _NEG, scores)

    ms = ms0
    mjs8 = jnp.zeros((G, n_nl), f32)
    idxc8 = jnp.zeros((G, n_nl), jnp.int32)
    for j in range(n_nl):
        mj = jnp.max(ms, axis=1, keepdims=True)
        cand = jnp.where(ms == mj, lane, big)
        i8 = jnp.min(cand, axis=1, keepdims=True)
        idxc8 = jnp.where(lane64 == j, i8, idxc8)
        mjs8 = jnp.where(lane64 == j, mj, mjs8)
        ms = jnp.where(lane == i8, _NEG, ms)
    acc8 = jnp.sum(jnp.exp(mjs8 - m8), axis=1, keepdims=True)

    for g in range(G):
        idxcol = idxc8[g:g + 1, :].reshape(n_nl, 1)
        oh = jnp.where(oh_lane == idxcol, 1.0, 0.0).astype(bf)
        rows_k = jax.lax.dot_general(oh, kb[g], (((1,), (0,)), ((), ())),
                                     preferred_element_type=f32)
        kg_ref[g, local_k:n_top, :] = rows_k

    norm = m8 + jnp.log(se8)
    remainder = jnp.log(1.0 - (p_loc + acc8 / se8)) + norm
    rem_ref[0] = jnp.broadcast_to(remainder, (G, 128))
    idx_ref[0] = idxc8


def _value_body(v_ref, vl_ref, mv_ref, *, S, D, local_k, G):
    for g in range(G):
        v2 = v_ref[g]
        mv_ref[0, g:g + 1, :] = jnp.sum(v2, axis=0, keepdims=True) * (1.0 / S)
        vl_ref[g, :, :] = v_ref[g, S - local_k:S, :]


def _sc_gather_rows(data2d, gidx, D):
    n_idx = gidx.shape[1]
    win = 128
    mesh = plsc.VectorSubcoreMesh(core_axis_name="core",
                                  subcore_axis_name="subcore")

    @functools.partial(pl.kernel,
                       out_type=jax.ShapeDtypeStruct((n_idx, D), data2d.dtype),
                       mesh=mesh)
    def gather_kernel(x_hbm, i_hbm, o_hbm):
        def body(i_vmem, o_vmem):
            pltpu.sync_copy(x_hbm.at[i_vmem.at[0]], o_vmem)

        pltpu.emit_pipeline(
            body,
            grid=(n_idx // win,),
            in_specs=[pl.BlockSpec((1, win), index_map=lambda i: (0, i))],
            out_specs=[pl.BlockSpec((win, D), index_map=lambda i: (i, 0))],
            core_axis_name="subcore",
            dimension_semantics=(pltpu.PARALLEL,),
        )(i_hbm, o_hbm)

    return gather_kernel(data2d, gidx)


def kernel(query, key, value, logmask, W):
    B, H, _, D = query.shape
    S = key.shape[2]
    BH = B * H
    R = W.shape[-1]
    G = _G
    NS = BH // G
    qp = jnp.matmul(query, W).reshape(BH, 1, R)
    qp = jnp.broadcast_to(qp, (BH, 8, R))
    k = key.reshape(BH, S, D)
    v = value.reshape(BH, S, D)

    sbody = functools.partial(_score_body, S=S, D=D,
                              local_k=_LOCAL_K, n_top=_K_TOP, G=G)
    kg, idx, rem = pl.pallas_call(
        sbody,
        grid=(NS,),
        in_specs=[
            pl.BlockSpec((G, 8, R), lambda i: (i, 0, 0)),
            pl.BlockSpec((G, D, R), lambda i: (i % (H // G), 0, 0)),
            pl.BlockSpec((G, S, D), lambda i: (i, 0, 0)),
        ],
        out_specs=[
            pl.BlockSpec((G, _K_TOP, D), lambda i: (i, 0, 0)),
            pl.BlockSpec((1, G, _K_TOP - _LOCAL_K), lambda i: (i, 0, 0)),
            pl.BlockSpec((1, G, D), lambda i: (i, 0, 0)),
        ],
        out_shape=[
            jax.ShapeDtypeStruct((BH, _K_TOP, D), jnp.float32),
            jax.ShapeDtypeStruct((NS, G, _K_TOP - _LOCAL_K), jnp.int32),
            jax.ShapeDtypeStruct((NS, G, D), jnp.float32),
        ],
        compiler_params=pltpu.CompilerParams(
            dimension_semantics=("arbitrary",)),
    )(qp, W, k)

    vbody = functools.partial(_value_body, S=S, D=D, local_k=_LOCAL_K, G=G)
    vl, mv = pl.pallas_call(
        vbody,
        grid=(NS,),
        in_specs=[
            pl.BlockSpec((G, S, D), lambda i: (i, 0, 0)),
        ],
        out_specs=[
            pl.BlockSpec((G, _LOCAL_K, D), lambda i: (i, 0, 0)),
            pl.BlockSpec((1, G, D), lambda i: (i, 0, 0)),
        ],
        out_shape=[
            jax.ShapeDtypeStruct((BH, _LOCAL_K, D), jnp.float32),
            jax.ShapeDtypeStruct((NS, G, D), jnp.float32),
        ],
        compiler_params=pltpu.CompilerParams(
            dimension_semantics=("arbitrary",)),
    )(v)

    n_nl = _K_TOP - _LOCAL_K
    gidx = (idx.reshape(BH, n_nl)
            + jnp.arange(BH, dtype=jnp.int32)[:, None] * S)
    v_nl = _sc_gather_rows(v.reshape(BH * S, D),
                           gidx.reshape(1, BH * n_nl), D)

    zeros_row = jnp.zeros((B, H, 1, D), jnp.float32)
    key_out = jnp.concatenate(
        [zeros_row, kg.reshape(B, H, _K_TOP, D)], axis=-2)
    value_out = jnp.concatenate(
        [mv.reshape(B, H, 1, D), vl.reshape(B, H, _LOCAL_K, D),
         v_nl.reshape(B, H, n_nl, D)], axis=-2)
    logmask_out = jnp.concatenate(
        [rem.reshape(B, H, 1, D)[..., :1],
         jnp.zeros((B, H, 1, _K_TOP), logmask.dtype)], axis=-1)
    return (query, key_out, value_out, logmask_out)


# speedup vs baseline: 1.0244x; 1.0244x over previous
import functools

import jax
import jax.numpy as jnp
from jax.experimental import pallas as pl
from jax.experimental.pallas import tpu as pltpu
from jax.experimental.pallas import tpu_sc as plsc

_K_TOP = 128
_LOCAL_K = 64
_G = 8
_NEG = -3.0e38


def _score_body(qp_ref, w_ref, k_ref, kg_ref, idx_ref, rem_ref,
                *, S, D, local_k, n_top, G):
    n_nl = n_top - local_k
    f32 = jnp.float32
    bf = jnp.bfloat16
    big = jnp.int32(2147483647)

    lane = jax.lax.broadcasted_iota(jnp.int32, (G, S), 1)
    is_local = lane >= (S - local_k)
    row = jax.lax.broadcasted_iota(jnp.int32, (G, S), 0)
    lane64 = jax.lax.broadcasted_iota(jnp.int32, (G, n_nl), 1)

    scores = None
    for g in range(G):
        Wb = w_ref[g].astype(bf)
        kbg = k_ref[g].astype(bf)
        kp = jax.lax.dot_general(kbg, Wb, (((1,), (0,)), ((), ())),
                                 preferred_element_type=f32)
        qpg = qp_ref[g].astype(bf)
        sf = jax.lax.dot_general(qpg, kp.astype(bf), (((1,), (1,)), ((), ())),
                                 preferred_element_type=f32)
        scores = sf if g == 0 else jnp.where(row == g, sf, scores)
        kg_ref[g, :, :] = k_ref[g, S - local_k:S, :]

    scores = scores * (D ** -0.5)
    m8 = jnp.max(scores, axis=1, keepdims=True)
    e8 = jnp.exp(scores - m8)
    se8 = jnp.sum(e8, axis=1, keepdims=True)
    p_loc = jnp.sum(jnp.where(is_local, e8, 0.0), axis=1,
                    keepdims=True) / se8
    ms0 = jnp.where(is_local, _NEG, scores)

    ms = ms0
    mjs8 = jnp.zeros((G, n_nl), f32)
    idxc8 = jnp.zeros((G, n_nl), jnp.int32)
    for j in range(n_nl):
        mj = jnp.max(ms, axis=1, keepdims=True)
        cand = jnp.where(ms == mj, lane, big)
        i8 = jnp.min(cand, axis=1, keepdims=True)
        idxc8 = jnp.where(lane64 == j, i8, idxc8)
        mjs8 = jnp.where(lane64 == j, mj, mjs8)
        ms = jnp.where(lane == i8, _NEG, ms)
    acc8 = jnp.sum(jnp.exp(mjs8 - m8), axis=1, keepdims=True)

    norm = m8 + jnp.log(se8)
    remainder = jnp.log(1.0 - (p_loc + acc8 / se8)) + norm
    rem_ref[0] = jnp.broadcast_to(remainder, (G, 128))
    idx_ref[0] = idxc8


def _value_body(v_ref, vl_ref, mv_ref, *, S, D, local_k, G):
    for g in range(G):
        v2 = v_ref[g]
        mv_ref[0, g:g + 1, :] = jnp.sum(v2, axis=0, keepdims=True) * (1.0 / S)
        vl_ref[g, :, :] = v_ref[g, S - local_k:S, :]


def _sc_gather_rows(data2d, gidx, D):
    n_idx = gidx.shape[1]
    win = 128
    mesh = plsc.VectorSubcoreMesh(core_axis_name="core",
                                  subcore_axis_name="subcore")

    @functools.partial(pl.kernel,
                       out_type=jax.ShapeDtypeStruct((n_idx, D), data2d.dtype),
                       mesh=mesh)
    def gather_kernel(x_hbm, i_hbm, o_hbm):
        def body(i_vmem, o_vmem):
            pltpu.sync_copy(x_hbm.at[i_vmem.at[0]], o_vmem)

        pltpu.emit_pipeline(
            body,
            grid=(n_idx // win,),
            in_specs=[pl.BlockSpec((1, win), index_map=lambda i: (0, i))],
            out_specs=[pl.BlockSpec((win, D), index_map=lambda i: (i, 0))],
            core_axis_name="subcore",
            dimension_semantics=(pltpu.PARALLEL,),
        )(i_hbm, o_hbm)

    return gather_kernel(data2d, gidx)


def kernel(query, key, value, logmask, W):
    B, H, _, D = query.shape
    S = key.shape[2]
    BH = B * H
    R = W.shape[-1]
    G = _G
    NS = BH // G
    qp = jnp.matmul(query, W).reshape(BH, 1, R)
    qp = jnp.broadcast_to(qp, (BH, 8, R))
    k = key.reshape(BH, S, D)
    v = value.reshape(BH, S, D)

    sbody = functools.partial(_score_body, S=S, D=D,
                              local_k=_LOCAL_K, n_top=_K_TOP, G=G)
    kg, idx, rem = pl.pallas_call(
        sbody,
        grid=(NS,),
        in_specs=[
            pl.BlockSpec((G, 8, R), lambda i: (i, 0, 0)),
            pl.BlockSpec((G, D, R), lambda i: (i % (H // G), 0, 0)),
            pl.BlockSpec((G, S, D), lambda i: (i, 0, 0)),
        ],
        out_specs=[
            pl.BlockSpec((G, _LOCAL_K, D), lambda i: (i, 0, 0)),
            pl.BlockSpec((1, G, _K_TOP - _LOCAL_K), lambda i: (i, 0, 0)),
            pl.BlockSpec((1, G, D), lambda i: (i, 0, 0)),
        ],
        out_shape=[
            jax.ShapeDtypeStruct((BH, _LOCAL_K, D), jnp.float32),
            jax.ShapeDtypeStruct((NS, G, _K_TOP - _LOCAL_K), jnp.int32),
            jax.ShapeDtypeStruct((NS, G, D), jnp.float32),
        ],
        compiler_params=pltpu.CompilerParams(
            dimension_semantics=("arbitrary",)),
    )(qp, W, k)

    vbody = functools.partial(_value_body, S=S, D=D, local_k=_LOCAL_K, G=G)
    vl, mv = pl.pallas_call(
        vbody,
        grid=(NS,),
        in_specs=[
            pl.BlockSpec((G, S, D), lambda i: (i, 0, 0)),
        ],
        out_specs=[
            pl.BlockSpec((G, _LOCAL_K, D), lambda i: (i, 0, 0)),
            pl.BlockSpec((1, G, D), lambda i: (i, 0, 0)),
        ],
        out_shape=[
            jax.ShapeDtypeStruct((BH, _LOCAL_K, D), jnp.float32),
            jax.ShapeDtypeStruct((NS, G, D), jnp.float32),
        ],
        compiler_params=pltpu.CompilerParams(
            dimension_semantics=("arbitrary",)),
    )(v)

    n_nl = _K_TOP - _LOCAL_K
    gidx = (idx.reshape(BH, n_nl)
            + jnp.arange(BH, dtype=jnp.int32)[:, None] * S)
    gidx = gidx.reshape(1, BH * n_nl)
    k_nl = _sc_gather_rows(k.reshape(BH * S, D), gidx, D)
    v_nl = _sc_gather_rows(v.reshape(BH * S, D), gidx, D)

    zeros_row = jnp.zeros((B, H, 1, D), jnp.float32)
    key_out = jnp.concatenate(
        [zeros_row, kg.reshape(B, H, _LOCAL_K, D),
         k_nl.reshape(B, H, n_nl, D)], axis=-2)
    value_out = jnp.concatenate(
        [mv.reshape(B, H, 1, D), vl.reshape(B, H, _LOCAL_K, D),
         v_nl.reshape(B, H, n_nl, D)], axis=-2)
    logmask_out = jnp.concatenate(
        [rem.reshape(B, H, 1, D)[..., :1],
         jnp.zeros((B, H, 1, _K_TOP), logmask.dtype)], axis=-1)
    return (query, key_out, value_out, logmask_out)
